# baseline (speedup 1.0000x reference)
import functools

import jax
import jax.numpy as jnp
from jax import lax
from jax.experimental import pallas as pl
from jax.experimental.pallas import tpu as pltpu
from jax.experimental.pallas import tpu_sc as plsc

N_NODES = 10000
H = 128
R = 8
E = 320000

NC = 2
NS = 16
NW = NC * NS
CHUNK = 128
CHUNKS_PER_W = -(-E // (NW * CHUNK))
CPW0 = 68
CPW1 = 2 * CHUNKS_PER_W - CPW0
E_PAD = NS * (CPW0 + CPW1) * CHUNK
N_ACC = 10112
ROWS_PER_TILE = N_ACC // NS


def _transform_body(h_ref, w_ref, out_ref):
    out_ref[0] = jnp.dot(h_ref[...], w_ref[0],
                         preferred_element_type=jnp.float32)


def _transform(h, rel_weight):
    bn = 2000
    return pl.pallas_call(
        _transform_body,
        grid=(N_NODES // bn, R),
        in_specs=[
            pl.BlockSpec((bn, H), lambda i, r: (i, 0)),
            pl.BlockSpec((1, H, H), lambda i, r: (r, 0, 0)),
        ],
        out_specs=pl.BlockSpec((1, bn, H), lambda i, r: (r, i, 0)),
        out_shape=jax.ShapeDtypeStruct((R, N_NODES, H), jnp.float32),
    )(h, rel_weight)


def _sc_scatter(table, gidx, dstp, zrows):
    mesh = plsc.VectorSubcoreMesh(core_axis_name="c", subcore_axis_name="s")

    @functools.partial(
        pl.kernel,
        mesh=mesh,
        out_type=jax.ShapeDtypeStruct((NC, N_ACC, H), jnp.float32),
        scratch_types=[
            pltpu.VMEM((CHUNK,), jnp.int32),
            pltpu.VMEM((CHUNK,), jnp.int32),
            pltpu.VMEM((CHUNK, H), jnp.float32),
            pltpu.VMEM_SHARED((N_ACC, H), jnp.float32),
            pltpu.SemaphoreType.DMA,
        ],
    )
    def k(table_hbm, gidx_hbm, dst_hbm, z_hbm, out_hbm,
          idx_v, dst_v, rows_v, acc, sem):
        c = lax.axis_index("c")
        s = lax.axis_index("s")
        base = jnp.where(c == 0, s * CPW0, NS * CPW0 + s * CPW1)
        n_my = jnp.where(c == 0, CPW0, CPW1)
        pltpu.sync_copy(z_hbm, acc.at[pl.ds(s * ROWS_PER_TILE, ROWS_PER_TILE)])
        plsc.subcore_barrier()

        def body(j, carry):
            chunk = base + j
            pltpu.sync_copy(gidx_hbm.at[chunk], idx_v)
            pltpu.sync_copy(dst_hbm.at[chunk], dst_v)
            pltpu.async_copy(table_hbm.at[idx_v], rows_v, sem).wait()
            pltpu.sync_copy(rows_v, acc.at[dst_v], add=True)
            return carry

        lax.fori_loop(0, n_my, body, 0)
        plsc.subcore_barrier()
        pltpu.sync_copy(acc.at[pl.ds(s * ROWS_PER_TILE, ROWS_PER_TILE)],
                        out_hbm.at[c, pl.ds(s * ROWS_PER_TILE, ROWS_PER_TILE)])

    return k(table, gidx, dstp, zrows)


def _epilogue_body(p0_ref, p1_ref, h_ref, lw_ref, b_ref, out_ref):
    out_ref[...] = (p0_ref[0] + p1_ref[0] + b_ref[...] +
                    jnp.dot(h_ref[...], lw_ref[...],
                            preferred_element_type=jnp.float32))


def _epilogue(partial, h, loop_weight, bias):
    bn = 2000
    return pl.pallas_call(
        _epilogue_body,
        grid=(N_NODES // bn,),
        in_specs=[
            pl.BlockSpec((1, bn, H), lambda i: (0, i, 0)),
            pl.BlockSpec((1, bn, H), lambda i: (1, i, 0)),
            pl.BlockSpec((bn, H), lambda i: (i, 0)),
            pl.BlockSpec((H, H), lambda i: (0, 0)),
            pl.BlockSpec((1, H), lambda i: (0, 0)),
        ],
        out_specs=pl.BlockSpec((bn, H), lambda i: (i, 0)),
        out_shape=jax.ShapeDtypeStruct((N_NODES, H), jnp.float32),
    )(partial, partial, h, loop_weight, bias.reshape(1, H))


def kernel(node_id, edge_index, edge_type, embedding, rel_weight,
           loop_weight, bias):
    h = jnp.take(embedding, node_id.astype(jnp.int32), axis=0)
    src = edge_index[0].astype(jnp.int32)
    dst = edge_index[1].astype(jnp.int32)
    et = edge_type.astype(jnp.int32)

    pad = E_PAD - E
    gidx = jnp.concatenate(
        [et * N_NODES + src, jnp.zeros((pad,), jnp.int32)]
    ).reshape(NW * CHUNKS_PER_W, CHUNK)
    dstp = jnp.concatenate(
        [dst, jnp.full((pad,), N_NODES, jnp.int32)]
    ).reshape(NW * CHUNKS_PER_W, CHUNK)
    zrows = jnp.zeros((ROWS_PER_TILE, H), jnp.float32)

    transformed = _transform(h, rel_weight).reshape(R * N_NODES, H)
    partial = _sc_scatter(transformed, gidx, dstp, zrows)
    return _epilogue(partial, h, loop_weight, bias)


# speedup vs baseline: 1.1527x; 1.1527x over previous
import functools

import jax
import jax.numpy as jnp
from jax import lax
from jax.experimental import pallas as pl
from jax.experimental.pallas import tpu as pltpu
from jax.experimental.pallas import tpu_sc as plsc

N_NODES = 10000
H = 128
R = 8
E = 320000

NC = 2
NS = 16
NW = NC * NS
CHUNK = 128
CHUNKS_PER_W = -(-E // (NW * CHUNK))
CPW0 = 99
CPW1 = 2 * CHUNKS_PER_W - CPW0
E_PAD = NS * (CPW0 + CPW1) * CHUNK
N_ACC = 10112
ROWS_PER_TILE = N_ACC // NS


def _transform_body(h_ref, w_ref, out_ref):
    out_ref[0] = jnp.dot(h_ref[...], w_ref[0],
                         preferred_element_type=jnp.float32)


def _transform(h, rel_weight):
    bn = 2000
    return pl.pallas_call(
        _transform_body,
        grid=(N_NODES // bn, R),
        in_specs=[
            pl.BlockSpec((bn, H), lambda i, r: (i, 0)),
            pl.BlockSpec((1, H, H), lambda i, r: (r, 0, 0)),
        ],
        out_specs=pl.BlockSpec((1, bn, H), lambda i, r: (r, i, 0)),
        out_shape=jax.ShapeDtypeStruct((R, N_NODES, H), jnp.float32),
    )(h, rel_weight)


def _sc_scatter(table, gidx, dstp, zrows):
    mesh = plsc.VectorSubcoreMesh(core_axis_name="c", subcore_axis_name="s")

    @functools.partial(
        pl.kernel,
        mesh=mesh,
        out_type=jax.ShapeDtypeStruct((NC, N_ACC, H), jnp.float32),
        scratch_types=[
            pltpu.VMEM((CHUNK,), jnp.int32),
            pltpu.VMEM((CHUNK,), jnp.int32),
            pltpu.VMEM((CHUNK, H), jnp.float32),
            pltpu.VMEM_SHARED((N_ACC, H), jnp.float32),
            pltpu.SemaphoreType.DMA,
        ],
    )
    def k(table_hbm, gidx_hbm, dst_hbm, z_hbm, out_hbm,
          idx_v, dst_v, rows_v, acc, sem):
        c = lax.axis_index("c")
        s = lax.axis_index("s")
        base = jnp.where(c == 0, s * CPW0, NS * CPW0 + s * CPW1)
        n_my = jnp.where(c == 0, CPW0, CPW1)
        pltpu.sync_copy(z_hbm, acc.at[pl.ds(s * ROWS_PER_TILE, ROWS_PER_TILE)])
        plsc.subcore_barrier()

        def body(j, carry):
            chunk = base + j
            pltpu.sync_copy(gidx_hbm.at[chunk], idx_v)
            pltpu.sync_copy(dst_hbm.at[chunk], dst_v)
            pltpu.async_copy(table_hbm.at[idx_v], rows_v, sem).wait()
            pltpu.sync_copy(rows_v, acc.at[dst_v], add=True)
            return carry

        lax.fori_loop(0, n_my, body, 0)
        plsc.subcore_barrier()
        pltpu.sync_copy(acc.at[pl.ds(s * ROWS_PER_TILE, ROWS_PER_TILE)],
                        out_hbm.at[c, pl.ds(s * ROWS_PER_TILE, ROWS_PER_TILE)])

    return k(table, gidx, dstp, zrows)


def _epilogue_body(p0_ref, p1_ref, h_ref, lw_ref, b_ref, out_ref):
    out_ref[...] = (p0_ref[0] + p1_ref[0] + b_ref[...] +
                    jnp.dot(h_ref[...], lw_ref[...],
                            preferred_element_type=jnp.float32))


def _epilogue(partial, h, loop_weight, bias):
    bn = 2000
    return pl.pallas_call(
        _epilogue_body,
        grid=(N_NODES // bn,),
        in_specs=[
            pl.BlockSpec((1, bn, H), lambda i: (0, i, 0)),
            pl.BlockSpec((1, bn, H), lambda i: (1, i, 0)),
            pl.BlockSpec((bn, H), lambda i: (i, 0)),
            pl.BlockSpec((H, H), lambda i: (0, 0)),
            pl.BlockSpec((1, H), lambda i: (0, 0)),
        ],
        out_specs=pl.BlockSpec((bn, H), lambda i: (i, 0)),
        out_shape=jax.ShapeDtypeStruct((N_NODES, H), jnp.float32),
    )(partial, partial, h, loop_weight, bias.reshape(1, H))


def kernel(node_id, edge_index, edge_type, embedding, rel_weight,
           loop_weight, bias):
    h = jnp.take(embedding, node_id.astype(jnp.int32), axis=0)
    src = edge_index[0].astype(jnp.int32)
    dst = edge_index[1].astype(jnp.int32)
    et = edge_type.astype(jnp.int32)

    pad = E_PAD - E
    gidx = jnp.concatenate(
        [et * N_NODES + src, jnp.zeros((pad,), jnp.int32)]
    ).reshape(NW * CHUNKS_PER_W, CHUNK)
    dstp = jnp.concatenate(
        [dst, jnp.full((pad,), N_NODES, jnp.int32)]
    ).reshape(NW * CHUNKS_PER_W, CHUNK)
    zrows = jnp.zeros((ROWS_PER_TILE, H), jnp.float32)

    transformed = _transform(h, rel_weight).reshape(R * N_NODES, H)
    partial = _sc_scatter(transformed, gidx, dstp, zrows)
    return _epilogue(partial, h, loop_weight, bias)
